# baseline (speedup 1.0000x reference)
import functools

import jax
import jax.numpy as jnp
from jax import lax
from jax.experimental import pallas as pl
from jax.experimental.pallas import tpu as pltpu
from jax.experimental.pallas import tpu_sc as plsc

NUM_CORES = 2
NUM_SUBCORES = 16
NUM_WORKERS = NUM_CORES * NUM_SUBCORES
LANES = 16
CHUNK = 128


def _build_sc_kernel(B, S, E, V):
    R = B * S
    rows_per_worker = R // NUM_WORKERS
    steps = rows_per_worker // CHUNK
    PAIR = 2 * E

    mesh = plsc.VectorSubcoreMesh(
        core_axis_name="c", subcore_axis_name="s",
        num_cores=NUM_CORES, num_subcores=NUM_SUBCORES)

    @functools.partial(
        pl.kernel,
        out_type=jax.ShapeDtypeStruct((R, E), jnp.float32),
        mesh=mesh,
        scratch_types=[
            pltpu.VMEM((CHUNK,), jnp.int32),
            pltpu.VMEM((CHUNK,), jnp.int32),
            pltpu.VMEM((CHUNK, PAIR), jnp.float32),
            pltpu.VMEM((CHUNK, E), jnp.float32),
            pltpu.VMEM((S, E), jnp.float32),
            pltpu.SemaphoreType.DMA,
        ],
    )
    def k(x1, table2, pos_hbm, out, idx_v, pidx_v, grows, res, pos_v, sem):
        c = lax.axis_index("c")
        s = lax.axis_index("s")
        wid = s * NUM_CORES + c
        base = wid * rows_per_worker
        pltpu.sync_copy(pos_hbm, pos_v)

        @pl.loop(0, steps)
        def _step(step):
            row0 = base + step * CHUNK
            pltpu.sync_copy(x1.at[pl.ds(row0, CHUNK)], idx_v)
            for g in range(CHUNK // LANES):
                sl = pl.ds(g * LANES, LANES)
                pidx_v[sl] = jax.lax.shift_right_logical(idx_v[sl], 1)
            pltpu.async_copy(table2.at[pidx_v], grows, sem).wait()
            s0 = lax.rem(row0, S)

            @pl.loop(0, CHUNK // LANES)
            def _grp(g):
                r0 = g * LANES
                hv = (idx_v[pl.ds(r0, LANES)] & 1) * E
                for i in range(LANES):
                    r = r0 + i
                    sr = lax.rem(s0 + r, S)
                    coff = hv[i]
                    for j in range(E // LANES):
                        res[r, pl.ds(j * LANES, LANES)] = (
                            grows[r, pl.ds(coff + j * LANES, LANES)]
                            + pos_v[sr, pl.ds(j * LANES, LANES)])

            pltpu.sync_copy(res, out.at[pl.ds(row0, CHUNK)])

    return k


def kernel(x, table, pos_encoding):
    B, S = x.shape
    V, E = table.shape
    x1 = x.astype(jnp.int32).reshape(-1)
    table2 = table.reshape(V // 2, 2 * E)
    pos_s = pos_encoding[0, :S, :]
    out = _build_sc_kernel(B, S, E, V)(x1, table2, pos_s)
    return out.reshape(B, S, E)


# speedup vs baseline: 1.0498x; 1.0498x over previous
import functools

import jax
import jax.numpy as jnp
from jax import lax
from jax.experimental import pallas as pl
from jax.experimental.pallas import tpu as pltpu
from jax.experimental.pallas import tpu_sc as plsc

NUM_CORES = 2
NUM_SUBCORES = 16
NUM_WORKERS = NUM_CORES * NUM_SUBCORES
LANES = 16
CHUNK = 128


def _build_sc_kernel(B, S, E, V):
    R = B * S
    RPW = R // NUM_WORKERS
    steps = RPW // CHUNK
    PAIR = 2 * E

    mesh = plsc.VectorSubcoreMesh(
        core_axis_name="c", subcore_axis_name="s",
        num_cores=NUM_CORES, num_subcores=NUM_SUBCORES)

    @functools.partial(
        pl.kernel,
        out_type=jax.ShapeDtypeStruct((R // 2, PAIR), jnp.float32),
        mesh=mesh,
        scratch_types=[
            pltpu.VMEM((RPW,), jnp.int32),
            pltpu.VMEM((2, CHUNK), jnp.int32),
            pltpu.VMEM((2, CHUNK, PAIR), jnp.float32),
            pltpu.VMEM((2, CHUNK // 2, PAIR), jnp.float32),
            pltpu.VMEM((S, E), jnp.float32),
            pltpu.SemaphoreType.DMA,
            pltpu.SemaphoreType.DMA,
            pltpu.SemaphoreType.DMA,
        ],
    )
    def k(x1, table2, pos_hbm, out2, idx_all, pidx, grows, res, pos_v,
          gsem0, gsem1, osem):
        c = lax.axis_index("c")
        s = lax.axis_index("s")
        wid = s * NUM_CORES + c
        base = pl.multiple_of(wid * RPW, RPW)
        gsems = (gsem0, gsem1)
        pltpu.sync_copy(pos_hbm, pos_v)
        pltpu.sync_copy(x1.at[pl.ds(base, RPW)], idx_all)

        def gather_start(t, b):
            for g in range(CHUNK // LANES):
                pidx[b, pl.ds(g * LANES, LANES)] = jax.lax.shift_right_logical(
                    idx_all[pl.ds(t * CHUNK + g * LANES, LANES)], 1)
            pltpu.async_copy(table2.at[pidx.at[b]], grows.at[b], gsems[b])

        def gather_wait(b):
            pltpu.make_async_copy(
                table2.at[pidx.at[b]], grows.at[b], gsems[b]).wait()

        def out_region(t):
            off = pl.multiple_of((base + t * CHUNK) // 2, CHUNK // 2)
            return out2.at[pl.ds(off, CHUNK // 2)]

        gather_start(0, 0)

        @pl.loop(0, steps // 2)
        def _pipe(it):
            t0 = it * 2
            for b in range(2):
                t = t0 + b
                gather_start(lax.rem(t + 1, steps), 1 - b)
                gather_wait(b)
                @pl.when(t >= 2)
                def _drain():
                    pltpu.make_async_copy(res.at[b], out_region(t - 2),
                                          osem).wait()
                s0 = lax.rem(base + t * CHUNK, S)

                @pl.loop(0, CHUNK // LANES)
                def _grp(g):
                    r0 = g * LANES
                    hv = (idx_all[pl.ds(t * CHUNK + r0, LANES)] & 1) * E
                    for i in range(LANES):
                        r = r0 + i
                        sr = lax.rem(s0 + r, S)
                        coff = hv[i]
                        half = (i % 2) * E
                        for j in range(E // LANES):
                            res[b, g * (LANES // 2) + i // 2,
                                pl.ds(half + j * LANES, LANES)] = (
                                grows[b, r, pl.ds(coff + j * LANES, LANES)]
                                + pos_v[sr, pl.ds(j * LANES, LANES)])

                pltpu.async_copy(res.at[b], out_region(t), osem)

        gather_wait(steps % 2)
        for t in (steps - 2, steps - 1):
            b = t % 2
            pltpu.make_async_copy(res.at[b], out_region(t), osem).wait()

    return k


def kernel(x, table, pos_encoding):
    B, S = x.shape
    V, E = table.shape
    x1 = x.astype(jnp.int32).reshape(-1)
    table2 = table.reshape(V // 2, 2 * E)
    pos_s = pos_encoding[0, :S, :]
    out2 = _build_sc_kernel(B, S, E, V)(x1, table2, pos_s)
    return out2.reshape(B, S, E)


# speedup vs baseline: 1.1769x; 1.1211x over previous
import functools

import jax
import jax.numpy as jnp
from jax import lax
from jax.experimental import pallas as pl
from jax.experimental.pallas import tpu as pltpu
from jax.experimental.pallas import tpu_sc as plsc

NUM_CORES = 2
NUM_SUBCORES = 16
NUM_WORKERS = NUM_CORES * NUM_SUBCORES
LANES = 16
IDX_CHUNK = 100


def _build_sc_kernel(B, S, E, V):
    BPW = B // NUM_WORKERS
    PAIRS = S // IDX_CHUNK

    mesh = plsc.VectorSubcoreMesh(
        core_axis_name="c", subcore_axis_name="s",
        num_cores=NUM_CORES, num_subcores=NUM_SUBCORES)

    @functools.partial(
        pl.kernel,
        out_type=jax.ShapeDtypeStruct((B, S, E), jnp.float32),
        mesh=mesh,
        scratch_types=[
            pltpu.VMEM((BPW * PAIRS, IDX_CHUNK), jnp.int32),
            pltpu.VMEM((2, S, E), jnp.float32),
            pltpu.VMEM((S, E), jnp.float32),
            pltpu.SemaphoreType.DMA,
            pltpu.SemaphoreType.DMA,
            pltpu.SemaphoreType.DMA,
        ],
        compiler_params=pltpu.CompilerParams(use_tc_tiling_on_sc=False),
    )
    def k(x2, table, pos_hbm, out3, idx_all, grows, pos_v,
          gsem0, gsem1, osem):
        c = lax.axis_index("c")
        s = lax.axis_index("s")
        wid = s * NUM_CORES + c
        base_b = pl.multiple_of(wid * BPW, BPW)
        gsems = (gsem0, gsem1)
        pltpu.sync_copy(pos_hbm, pos_v)
        pltpu.sync_copy(x2.at[pl.ds(base_b * PAIRS, BPW * PAIRS)], idx_all)

        def gather_start(t, b):
            for p in range(PAIRS):
                pltpu.async_copy(
                    table.at[idx_all.at[t * PAIRS + p]],
                    grows.at[b, pl.ds(p * IDX_CHUNK, IDX_CHUNK)],
                    gsems[b])

        def gather_wait(t, b):
            for p in range(PAIRS):
                pltpu.make_async_copy(
                    table.at[idx_all.at[t * PAIRS + p]],
                    grows.at[b, pl.ds(p * IDX_CHUNK, IDX_CHUNK)],
                    gsems[b]).wait()

        def out_write(t, b):
            return pltpu.async_copy(grows.at[b], out3.at[base_b + t], osem)

        def out_drain(t, b):
            pltpu.make_async_copy(grows.at[b], out3.at[base_b + t],
                                  osem).wait()

        gather_start(0, 0)

        @pl.loop(0, BPW // 2)
        def _pipe(it):
            t0 = it * 2
            for b in range(2):
                t = t0 + b
                @pl.when(t >= 1)
                def _drain():
                    out_drain(t - 1, 1 - b)

                gather_start(lax.rem(t + 1, BPW), 1 - b)
                gather_wait(t, b)

                @pl.loop(0, S)
                def _row(r):
                    for j in range(E // LANES):
                        sl = pl.ds(j * LANES, LANES)
                        grows[b, r, sl] = grows[b, r, sl] + pos_v[r, sl]

                out_write(t, b)

        gather_wait(0, BPW % 2)
        out_drain(BPW - 1, (BPW - 1) % 2)

    return k


def kernel(x, table, pos_encoding):
    B, S = x.shape
    V, E = table.shape
    x2 = x.astype(jnp.int32).reshape(-1, IDX_CHUNK)
    pos_s = pos_encoding[0, :S, :]
    return _build_sc_kernel(B, S, E, V)(x2, table, pos_s)
